# baseline (speedup 1.0000x reference)
import functools
import math

import jax
import jax.numpy as jnp
from jax import lax
from jax.experimental import pallas as pl
from jax.experimental.pallas import tpu as pltpu
from jax.experimental.pallas import tpu_sc as plsc

B, S, D = 1024, 50, 128
N = B * S
V = 100000
NB = 100
MV = 10
HID = 128
D3 = D // 3

NC, NS = 2, 16
NW = NC * NS
TPW = N // NW
CH = 8
ROWS = CH * MV
GSZ = 80
NGS = ROWS // GSZ
STR_ROWS = 10000
TLOAD = 640


def _sc_gather(rel_idx, str_idx, rel_table, str_table, zeros, scidx_all):
    n = rel_idx.shape[0]
    tpw = n // NW
    nchunk = tpw // CH
    mesh = plsc.VectorSubcoreMesh(core_axis_name="c", subcore_axis_name="s",
                                  num_cores=NC, num_subcores=NS)

    @functools.partial(
        pl.kernel,
        out_type=(jax.ShapeDtypeStruct((n, D), jnp.float32),
                  jax.ShapeDtypeStruct((n, D), jnp.float32)),
        mesh=mesh,
        compiler_params=pltpu.CompilerParams(use_tc_tiling_on_sc=True),
        scratch_types=[
            pltpu.VMEM((tpw,), jnp.int32),
            pltpu.VMEM((tpw * MV,), jnp.int32),
            pltpu.VMEM((CH, D), jnp.float32),
            pltpu.VMEM((CH, D), jnp.float32),
            pltpu.VMEM((ROWS, D), jnp.float32),
            pltpu.VMEM((ROWS, D), jnp.float32),
            pltpu.VMEM((2, NGS, GSZ), jnp.int32),
            pltpu.VMEM((CH, D), jnp.float32),
            pltpu.VMEM_SHARED((NS * 2 * CH, D), jnp.float32),
            pltpu.VMEM_SHARED((STR_ROWS, D), jnp.float32),
            pltpu.SemaphoreType.DMA, pltpu.SemaphoreType.DMA,
            pltpu.SemaphoreType.DMA, pltpu.SemaphoreType.DMA,
            pltpu.SemaphoreType.DMA, pltpu.SemaphoreType.DMA,
            pltpu.SemaphoreType.DMA, pltpu.SemaphoreType.DMA,
            pltpu.SemaphoreType.DMA, pltpu.SemaphoreType.DMA,
        ],
    )
    def k(rel_idx_hbm, str_idx_hbm, rel_tab_hbm, str_tab_hbm, zeros_hbm,
          scidx_hbm, rel_out, str_out,
          ridx_all, sidx_all, rrow0, rrow1, srow0, srow1, scidx_v, zero_v,
          acc_sh, stab_sh, rsem0, rsem1, ssem0, ssem1, asem0, asem1,
          wsem0, wsem1, osem0, osem1):
        cid = lax.axis_index("c")
        sid = lax.axis_index("s")
        wid = sid * NC + cid
        base0 = wid * tpw

        rrow = (rrow0, rrow1)
        srow = (srow0, srow1)
        rsem = (rsem0, rsem1)
        ssem = (ssem0, ssem1)
        asem = (asem0, asem1)
        wsem = (wsem0, wsem1)
        osem = (osem0, osem1)

        pltpu.sync_copy(zeros_hbm, zero_v)
        pltpu.sync_copy(scidx_hbm.at[sid], scidx_v)
        pltpu.sync_copy(rel_idx_hbm.at[pl.ds(base0, tpw)], ridx_all)
        pltpu.sync_copy(str_idx_hbm.at[pl.ds(base0 * MV, tpw * MV)], sidx_all)
        trows = min(TLOAD, STR_ROWS - (NS - 1) * TLOAD)
        @pl.when(sid < NS - 1)
        def _():
            pltpu.sync_copy(str_tab_hbm.at[pl.ds(sid * TLOAD, TLOAD)],
                            stab_sh.at[pl.ds(sid * TLOAD, TLOAD)])
        @pl.when(sid == NS - 1)
        def _():
            pltpu.sync_copy(str_tab_hbm.at[pl.ds((NS - 1) * TLOAD, trows)],
                            stab_sh.at[pl.ds((NS - 1) * TLOAD, trows)])
        plsc.subcore_barrier()

        def rel_gather_args(c, b):
            off = c * CH
            return (rel_tab_hbm.at[ridx_all.at[pl.ds(off, CH)]],
                    rrow[b], rsem[b])

        def str_gather_args(c, b, j):
            off = c * ROWS + j * GSZ
            return (stab_sh.at[sidx_all.at[pl.ds(off, GSZ)]],
                    srow[b].at[pl.ds(j * GSZ, GSZ)], ssem[b])

        def add_args(b, j):
            return (srow[b].at[pl.ds(j * GSZ, GSZ)],
                    acc_sh.at[scidx_v.at[b, j]], asem[b])

        def rel_out_args(c, b):
            return (rrow[b], rel_out.at[pl.ds(base0 + c * CH, CH)], wsem[b])

        def acc_region(b):
            return acc_sh.at[pl.ds((sid * 2 + b) * CH, CH)]

        def str_out_args(c, b):
            return (acc_region(b), str_out.at[pl.ds(base0 + c * CH, CH)],
                    osem[b])

        def issue_gathers(c, b):
            pltpu.async_copy(*rel_gather_args(c, b))
            for j in range(NGS):
                pltpu.async_copy(*str_gather_args(c, b, j))

        def issue(c, b):
            pltpu.make_async_copy(*rel_out_args(c, b)).wait()
            issue_gathers(c, b)

        def process(c, b):
            pltpu.make_async_copy(*rel_gather_args(c, b)).wait()
            pltpu.async_copy(*rel_out_args(c, b))
            if isinstance(c, int):
                if c >= 2:
                    pltpu.make_async_copy(*str_out_args(c, b)).wait()
            else:
                @pl.when(c >= 2)
                def _():
                    pltpu.make_async_copy(*str_out_args(c, b)).wait()
            pltpu.sync_copy(zero_v, acc_region(b))
            for j in range(NGS):
                pltpu.make_async_copy(*str_gather_args(c, b, j)).wait()
            adds = [pltpu.async_copy(*add_args(b, j), add=True)
                    for j in range(NGS)]
            for a in adds:
                a.wait()
            pltpu.async_copy(*str_out_args(c, b))

        issue_gathers(0, 0)
        issue_gathers(1, 1)

        def body(cc, carry):
            c = cc * 2
            process(c, 0)
            process(c + 1, 1)
            issue(c + 2, 0)
            issue(c + 3, 1)
            return carry

        lax.fori_loop(0, (nchunk - 2) // 2, body, 0)
        if nchunk % 2 == 0:
            tail = [nchunk - 2, nchunk - 1]
            process(nchunk - 2, 0)
            process(nchunk - 1, 1)
        else:
            tail = [nchunk - 2, nchunk - 1]
            process(nchunk - 3, (nchunk - 3) % 2)
            issue(nchunk - 1, (nchunk - 1) % 2)
            process(nchunk - 2, (nchunk - 2) % 2)
            process(nchunk - 1, (nchunk - 1) % 2)
        for c in tail:
            pltpu.make_async_copy(*rel_out_args(c, c % 2)).wait()
            pltpu.make_async_copy(*str_out_args(c, c % 2)).wait()

    return k(rel_idx, str_idx, rel_table, str_table, zeros, scidx_all)


TBLK = 400
GRID = N // TBLK

_LOG_MAX = math.log(1e6)


def _quant(x):
    c = jnp.maximum(x, 1.0)
    lt = jnp.log(c) / _LOG_MAX * (NB - 1)
    return jnp.clip(lt.astype(jnp.int32), 0, NB - 1)


BB = TBLK // S


def _tc_body(ts_ref, num_ref, tind_ref, rel_ref, ssum_ref,
             Cat_ref, T4_ref, Wna_ref, W1b_ref, W2_ref,
             b1_ref, pe_ref, E_ref, P4_ref, ONES4_ref, IL_ref, out_ref):
    f32, bf16 = jnp.float32, jnp.bfloat16
    ts = ts_ref[...]
    rel_t = jnp.concatenate(
        [jnp.zeros((BB, 1), f32), ts[:, 1:] - ts[:, :-1]], axis=1)
    sess_t = ts - ts[:, :1]
    dot = functools.partial(jnp.dot, preferred_element_type=f32)
    X4 = jnp.concatenate(
        [_quant(ts), _quant(rel_t), _quant(sess_t), tind_ref[...]],
        axis=1).astype(bf16)
    Y4 = dot(E_ref[...], X4).astype(bf16)
    Z = dot(Y4 * P4_ref[...], ONES4_ref[...])
    ohall = (Z == IL_ref[...]).astype(bf16)
    oh3 = ohall[:, :3 * 128]
    oht = ohall[:, 3 * 128:]
    temporal = dot(oh3, Cat_ref[...])
    xnum = num_ref[...]
    xnum = jnp.concatenate(
        [xnum, jnp.zeros((BB, S, 128 - MV), f32)], axis=2)
    xnum = xnum.astype(bf16).reshape(TBLK, 128)
    pre = (dot(xnum, Wna_ref[...])
           + dot((ssum_ref[...] * (1.0 / MV)).astype(bf16), W1b_ref[...])
           + dot(oht, T4_ref[...]) + b1_ref[...])
    val = dot(jnp.maximum(pre, 0.0).astype(bf16), W2_ref[...])
    res = rel_ref[...] + temporal + val + pe_ref[...]
    out_ref[...] = res.reshape(BB, S, D)


def _tc_call(ts, xnum, tind, rel_rows, str_sum,
             Cat, T4, Wna, W1b, W2, b1f, pe_fb, Esel, P4, ONES4, IL,
             blk0, prev_out=None):
    n_tok = rel_rows.shape[0]
    grid = n_tok // TBLK
    full = lambda a, b: pl.BlockSpec((a, b), lambda i: (0, 0))
    in_specs = [
        pl.BlockSpec((BB, S), lambda i: (i, 0)),
        pl.BlockSpec((BB, S, MV), lambda i: (i, 0, 0)),
        pl.BlockSpec((BB, S), lambda i: (i, 0)),
        pl.BlockSpec((TBLK, D), lambda i: (i, 0)),
        pl.BlockSpec((TBLK, D), lambda i: (i, 0)),
        full(3 * 128, D), full(128, D), full(128, D),
        full(HID, D), full(D, D),
        full(1, D), full(TBLK, D),
        full(TBLK, BB), full(TBLK, 4 * S), full(4 * S, 512),
        full(TBLK, 512),
    ]
    args = [ts, xnum, tind, rel_rows, str_sum,
            Cat, T4, Wna, W1b, W2, b1f, pe_fb, Esel, P4, ONES4, IL]
    kw = {}
    body = _tc_body
    if prev_out is not None:
        in_specs.append(pl.BlockSpec(memory_space=pl.ANY))
        args.append(prev_out)
        kw['input_output_aliases'] = {len(args) - 1: 0}
        body = lambda *refs: _tc_body(*refs[:16], refs[17])
    return pl.pallas_call(
        body,
        grid=(grid,),
        in_specs=in_specs,
        out_specs=pl.BlockSpec((BB, S, D), lambda i: (i + blk0, 0, 0)),
        out_shape=jax.ShapeDtypeStruct((B, S, D), jnp.float32),
        **kw,
    )(*args)


def kernel(relation_ids, timestamps, numeric_values, string_hashes,
           type_indicators, relation_table, abs_tab, rel_tab, sess_tab,
           Wp, bp, Wn, bn, string_tab, type_tab, W1, b1, W2, b2, pe):
    f32 = jnp.float32
    bf16 = jnp.bfloat16
    ts = timestamps.astype(f32)

    ridx = relation_ids.reshape(N).astype(jnp.int32)
    sidx = string_hashes.reshape(N * MV).astype(jnp.int32)
    zeros = jnp.zeros((CH, D), f32)
    r_ids = jnp.arange(ROWS, dtype=jnp.int32) // MV
    slot = jnp.arange(2, dtype=jnp.int32)[:, None] * CH
    sidb = jnp.arange(NS, dtype=jnp.int32)[:, None, None] * (2 * CH)
    scidx_all = (sidb + slot[None] + r_ids[None, None, :]
                 ).reshape(NS, 2, NGS, GSZ)
    HB = B // 2
    HT = HB * S
    rtab = relation_table.astype(f32)
    stab = string_tab.astype(f32)
    rel1, str1 = _sc_gather(ridx[:HT], sidx[:HT * MV], rtab, stab,
                            zeros, scidx_all)
    rel2, str2 = _sc_gather(ridx[HT:], sidx[HT * MV:], rtab, stab,
                            zeros, scidx_all)

    A = abs_tab @ Wp[:D3]
    R = rel_tab @ Wp[D3:2 * D3]
    Se = sess_tab @ Wp[2 * D3:]
    zpad = lambda t: jnp.concatenate(
        [t, jnp.zeros((128 - t.shape[0], D), f32)], axis=0)
    Cat = jnp.concatenate([zpad(A), zpad(R), zpad(Se)], axis=0).astype(bf16)
    W1a, W1b, W1c = W1[:HID], W1[HID:2 * HID], W1[2 * HID:]
    Wna = zpad(Wn @ W1a).astype(bf16)
    T4 = zpad(type_tab @ W1c).astype(bf16)
    b1f = (b1 + bn @ W1a).reshape(1, D)
    pe_fb = jnp.tile(pe[:S], (TBLK // S, 1)) + (bp + b2)[None, :]
    rr = jnp.arange(TBLK, dtype=jnp.int32)
    Esel = (rr[:, None] // S == jnp.arange(BB)[None, :]).astype(bf16)
    l4 = jnp.arange(4 * S)
    P4 = (rr[:, None] % S == (l4 % S)[None, :]).astype(bf16)
    c512 = jnp.arange(512)
    ONES4 = ((l4[:, None] // S) == (c512 // 128)[None, :]).astype(bf16)
    IL = jnp.broadcast_to((c512 % 128)[None, :], (TBLK, 512)).astype(f32)

    tind = type_indicators.astype(jnp.int32)
    consts = (Cat, T4, Wna, W1b.astype(bf16), W2.astype(bf16),
              b1f, pe_fb, Esel, P4, ONES4, IL)
    out1 = _tc_call(ts[:HB], numeric_values[:HB], tind[:HB], rel1, str1,
                    *consts, blk0=0)
    return _tc_call(ts[HB:], numeric_values[HB:], tind[HB:], rel2, str2,
                    *consts, blk0=HB // BB, prev_out=out1)


# speedup vs baseline: 1.2575x; 1.2575x over previous
import functools
import math

import jax
import jax.numpy as jnp
from jax import lax
from jax.experimental import pallas as pl
from jax.experimental.pallas import tpu as pltpu
from jax.experimental.pallas import tpu_sc as plsc

B, S, D = 1024, 50, 128
N = B * S
V = 100000
NB = 100
MV = 10
HID = 128
D3 = D // 3

NC, NS = 2, 16
NW = NC * NS
TPW = N // NW
CH = 32
NCHUNK = TPW // CH
ROWS = CH * MV
GSZ = 64
NGS = ROWS // GSZ


def _sc_gather(rel_idx, str_idx, rel_table, str_table, zeros, scidx_all):
    n = rel_idx.shape[0]
    tpw = n // NW
    nchunk = tpw // CH
    mesh = plsc.VectorSubcoreMesh(core_axis_name="c", subcore_axis_name="s",
                                  num_cores=NC, num_subcores=NS)

    @functools.partial(
        pl.kernel,
        out_type=(jax.ShapeDtypeStruct((n, D), jnp.float32),
                  jax.ShapeDtypeStruct((n, D), jnp.float32)),
        mesh=mesh,
        compiler_params=pltpu.CompilerParams(use_tc_tiling_on_sc=True),
        scratch_types=[
            pltpu.VMEM((tpw,), jnp.int32),
            pltpu.VMEM((tpw * MV,), jnp.int32),
            pltpu.VMEM((CH, D), jnp.float32),
            pltpu.VMEM((CH, D), jnp.float32),
            pltpu.VMEM((ROWS, D), jnp.float32),
            pltpu.VMEM((ROWS, D), jnp.float32),
            pltpu.VMEM((2, NGS, GSZ), jnp.int32),
            pltpu.VMEM((CH, D), jnp.float32),
            pltpu.VMEM_SHARED((NS * 2 * CH, D), jnp.float32),
            pltpu.SemaphoreType.DMA, pltpu.SemaphoreType.DMA,
            pltpu.SemaphoreType.DMA, pltpu.SemaphoreType.DMA,
            pltpu.SemaphoreType.DMA, pltpu.SemaphoreType.DMA,
            pltpu.SemaphoreType.DMA, pltpu.SemaphoreType.DMA,
            pltpu.SemaphoreType.DMA, pltpu.SemaphoreType.DMA,
        ],
    )
    def k(rel_idx_hbm, str_idx_hbm, rel_tab_hbm, str_tab_hbm, zeros_hbm,
          scidx_hbm, rel_out, str_out,
          ridx_all, sidx_all, rrow0, rrow1, srow0, srow1, scidx_v, zero_v,
          acc_sh, rsem0, rsem1, ssem0, ssem1, asem0, asem1,
          wsem0, wsem1, osem0, osem1):
        cid = lax.axis_index("c")
        sid = lax.axis_index("s")
        wid = sid * NC + cid
        base0 = wid * tpw

        rrow = (rrow0, rrow1)
        srow = (srow0, srow1)
        rsem = (rsem0, rsem1)
        ssem = (ssem0, ssem1)
        asem = (asem0, asem1)
        wsem = (wsem0, wsem1)
        osem = (osem0, osem1)

        pltpu.sync_copy(zeros_hbm, zero_v)
        pltpu.sync_copy(scidx_hbm.at[sid], scidx_v)
        pltpu.sync_copy(rel_idx_hbm.at[pl.ds(base0, tpw)], ridx_all)
        pltpu.sync_copy(str_idx_hbm.at[pl.ds(base0 * MV, tpw * MV)], sidx_all)

        def rel_gather_args(c, b):
            off = c * CH
            return (rel_tab_hbm.at[ridx_all.at[pl.ds(off, CH)]],
                    rrow[b], rsem[b])

        def str_gather_args(c, b, j):
            off = c * ROWS + j * GSZ
            return (str_tab_hbm.at[sidx_all.at[pl.ds(off, GSZ)]],
                    srow[b].at[pl.ds(j * GSZ, GSZ)], ssem[b])

        def add_args(b, j):
            return (srow[b].at[pl.ds(j * GSZ, GSZ)],
                    acc_sh.at[scidx_v.at[b, j]], asem[b])

        def rel_out_args(c, b):
            return (rrow[b], rel_out.at[pl.ds(base0 + c * CH, CH)], wsem[b])

        def acc_region(b):
            return acc_sh.at[pl.ds((sid * 2 + b) * CH, CH)]

        def str_out_args(c, b):
            return (acc_region(b), str_out.at[pl.ds(base0 + c * CH, CH)],
                    osem[b])

        def issue_gathers(c, b):
            pltpu.async_copy(*rel_gather_args(c, b))
            for j in range(NGS):
                pltpu.async_copy(*str_gather_args(c, b, j))

        def issue(c, b):
            pltpu.make_async_copy(*rel_out_args(c, b)).wait()
            issue_gathers(c, b)

        def process(c, b):
            pltpu.make_async_copy(*rel_gather_args(c, b)).wait()
            pltpu.async_copy(*rel_out_args(c, b))
            if isinstance(c, int):
                if c >= 2:
                    pltpu.make_async_copy(*str_out_args(c, b)).wait()
            else:
                @pl.when(c >= 2)
                def _():
                    pltpu.make_async_copy(*str_out_args(c, b)).wait()
            pltpu.sync_copy(zero_v, acc_region(b))
            for j in range(NGS):
                pltpu.make_async_copy(*str_gather_args(c, b, j)).wait()
            adds = [pltpu.async_copy(*add_args(b, j), add=True)
                    for j in range(NGS)]
            for a in adds:
                a.wait()
            pltpu.async_copy(*str_out_args(c, b))

        issue_gathers(0, 0)
        issue_gathers(1, 1)

        def body(cc, carry):
            c = cc * 2
            process(c, 0)
            process(c + 1, 1)
            issue(c + 2, 0)
            issue(c + 3, 1)
            return carry

        lax.fori_loop(0, (nchunk - 2) // 2, body, 0)
        if nchunk % 2 == 0:
            tail = [nchunk - 2, nchunk - 1]
            process(nchunk - 2, 0)
            process(nchunk - 1, 1)
        else:
            tail = [nchunk - 2, nchunk - 1]
            process(nchunk - 3, (nchunk - 3) % 2)
            issue(nchunk - 1, (nchunk - 1) % 2)
            process(nchunk - 2, (nchunk - 2) % 2)
            process(nchunk - 1, (nchunk - 1) % 2)
        for c in tail:
            pltpu.make_async_copy(*rel_out_args(c, c % 2)).wait()
            pltpu.make_async_copy(*str_out_args(c, c % 2)).wait()

    return k(rel_idx, str_idx, rel_table, str_table, zeros, scidx_all)


TBLK = 400
GRID = N // TBLK

_LOG_MAX = math.log(1e6)


def _quant(x):
    c = jnp.maximum(x, 1.0)
    lt = jnp.log(c) / _LOG_MAX * (NB - 1)
    return jnp.clip(lt.astype(jnp.int32), 0, NB - 1)


BB = TBLK // S


def _tc_body(ts_ref, num_ref, tind_ref, rel_ref, ssum_ref,
             Cat_ref, T4_ref, Wna_ref, W1b_ref, W2_ref,
             b1_ref, pe_ref, E_ref, P4_ref, ONES4_ref, IL_ref, out_ref):
    f32, bf16 = jnp.float32, jnp.bfloat16
    ts = ts_ref[...]
    rel_t = jnp.concatenate(
        [jnp.zeros((BB, 1), f32), ts[:, 1:] - ts[:, :-1]], axis=1)
    sess_t = ts - ts[:, :1]
    dot = functools.partial(jnp.dot, preferred_element_type=f32)
    X4 = jnp.concatenate(
        [_quant(ts), _quant(rel_t), _quant(sess_t), tind_ref[...]],
        axis=1).astype(bf16)
    Y4 = dot(E_ref[...], X4).astype(bf16)
    Z = dot(Y4 * P4_ref[...], ONES4_ref[...])
    ohall = (Z == IL_ref[...]).astype(bf16)
    oh3 = ohall[:, :3 * 128]
    oht = ohall[:, 3 * 128:]
    temporal = dot(oh3, Cat_ref[...])
    xnum = num_ref[...]
    xnum = jnp.concatenate(
        [xnum, jnp.zeros((BB, S, 128 - MV), f32)], axis=2)
    xnum = xnum.astype(bf16).reshape(TBLK, 128)
    pre = (dot(xnum, Wna_ref[...])
           + dot((ssum_ref[...] * (1.0 / MV)).astype(bf16), W1b_ref[...])
           + dot(oht, T4_ref[...]) + b1_ref[...])
    val = dot(jnp.maximum(pre, 0.0).astype(bf16), W2_ref[...])
    res = rel_ref[...] + temporal + val + pe_ref[...]
    out_ref[...] = res.reshape(BB, S, D)


def _tc_call(ts, xnum, tind, rel_rows, str_sum,
             Cat, T4, Wna, W1b, W2, b1f, pe_fb, Esel, P4, ONES4, IL,
             blk0, prev_out=None):
    n_tok = rel_rows.shape[0]
    grid = n_tok // TBLK
    full = lambda a, b: pl.BlockSpec((a, b), lambda i: (0, 0))
    in_specs = [
        pl.BlockSpec((BB, S), lambda i: (i, 0)),
        pl.BlockSpec((BB, S, MV), lambda i: (i, 0, 0)),
        pl.BlockSpec((BB, S), lambda i: (i, 0)),
        pl.BlockSpec((TBLK, D), lambda i: (i, 0)),
        pl.BlockSpec((TBLK, D), lambda i: (i, 0)),
        full(3 * 128, D), full(128, D), full(128, D),
        full(HID, D), full(D, D),
        full(1, D), full(TBLK, D),
        full(TBLK, BB), full(TBLK, 4 * S), full(4 * S, 512),
        full(TBLK, 512),
    ]
    args = [ts, xnum, tind, rel_rows, str_sum,
            Cat, T4, Wna, W1b, W2, b1f, pe_fb, Esel, P4, ONES4, IL]
    kw = {}
    body = _tc_body
    if prev_out is not None:
        in_specs.append(pl.BlockSpec(memory_space=pl.ANY))
        args.append(prev_out)
        kw['input_output_aliases'] = {len(args) - 1: 0}
        body = lambda *refs: _tc_body(*refs[:16], refs[17])
    return pl.pallas_call(
        body,
        grid=(grid,),
        in_specs=in_specs,
        out_specs=pl.BlockSpec((BB, S, D), lambda i: (i + blk0, 0, 0)),
        out_shape=jax.ShapeDtypeStruct((B, S, D), jnp.float32),
        **kw,
    )(*args)


def kernel(relation_ids, timestamps, numeric_values, string_hashes,
           type_indicators, relation_table, abs_tab, rel_tab, sess_tab,
           Wp, bp, Wn, bn, string_tab, type_tab, W1, b1, W2, b2, pe):
    f32 = jnp.float32
    bf16 = jnp.bfloat16
    ts = timestamps.astype(f32)

    ridx = relation_ids.reshape(N).astype(jnp.int32)
    sidx = string_hashes.reshape(N * MV).astype(jnp.int32)
    zeros = jnp.zeros((CH, D), f32)
    r_ids = jnp.arange(ROWS, dtype=jnp.int32) // MV
    slot = jnp.arange(2, dtype=jnp.int32)[:, None] * CH
    sidb = jnp.arange(NS, dtype=jnp.int32)[:, None, None] * (2 * CH)
    scidx_all = (sidb + slot[None] + r_ids[None, None, :]
                 ).reshape(NS, 2, NGS, GSZ)
    HB = B // 2
    HT = HB * S
    rtab = relation_table.astype(f32)
    stab = string_tab.astype(f32)
    rel1, str1 = _sc_gather(ridx[:HT], sidx[:HT * MV], rtab, stab,
                            zeros, scidx_all)
    rel2, str2 = _sc_gather(ridx[HT:], sidx[HT * MV:], rtab, stab,
                            zeros, scidx_all)

    A = abs_tab @ Wp[:D3]
    R = rel_tab @ Wp[D3:2 * D3]
    Se = sess_tab @ Wp[2 * D3:]
    zpad = lambda t: jnp.concatenate(
        [t, jnp.zeros((128 - t.shape[0], D), f32)], axis=0)
    Cat = jnp.concatenate([zpad(A), zpad(R), zpad(Se)], axis=0).astype(bf16)
    W1a, W1b, W1c = W1[:HID], W1[HID:2 * HID], W1[2 * HID:]
    Wna = zpad(Wn @ W1a).astype(bf16)
    T4 = zpad(type_tab @ W1c).astype(bf16)
    b1f = (b1 + bn @ W1a).reshape(1, D)
    pe_fb = jnp.tile(pe[:S], (TBLK // S, 1)) + (bp + b2)[None, :]
    rr = jnp.arange(TBLK, dtype=jnp.int32)
    Esel = (rr[:, None] // S == jnp.arange(BB)[None, :]).astype(bf16)
    l4 = jnp.arange(4 * S)
    P4 = (rr[:, None] % S == (l4 % S)[None, :]).astype(bf16)
    c512 = jnp.arange(512)
    ONES4 = ((l4[:, None] // S) == (c512 // 128)[None, :]).astype(bf16)
    IL = jnp.broadcast_to((c512 % 128)[None, :], (TBLK, 512)).astype(f32)

    tind = type_indicators.astype(jnp.int32)
    consts = (Cat, T4, Wna, W1b.astype(bf16), W2.astype(bf16),
              b1f, pe_fb, Esel, P4, ONES4, IL)
    out1 = _tc_call(ts[:HB], numeric_values[:HB], tind[:HB], rel1, str1,
                    *consts, blk0=0)
    return _tc_call(ts[HB:], numeric_values[HB:], tind[HB:], rel2, str2,
                    *consts, blk0=HB // BB, prev_out=out1)
